# Initial kernel scaffold; baseline (speedup 1.0000x reference)
import jax
import jax.numpy as jnp
from jax.experimental import pallas as pl


def kernel(x, edge_index, edge_attr, batch, root_mask, params):
    raise NotImplementedError("write your pallas kernel here")


# speedup vs baseline: 4.2696x; 4.2696x over previous
import functools

import jax
import jax.numpy as jnp
from jax import lax
from jax.experimental import pallas as pl
from jax.experimental.pallas import tpu as pltpu
from jax.experimental.pallas import tpu_sc as plsc

N = 10000
EMB = 128
E = 320000
NTYPES = 4

NC, NS = 2, 16
NW = NC * NS
CH = 80
EPW = E // NW
NCHUNK = EPW // CH
NPAD = 10240
EMB_PW = NPAD // NW
NAPAD = 10112
ROWS_PT = NAPAD // NS

_mesh = plsc.VectorSubcoreMesh(core_axis_name="c", subcore_axis_name="s")


@functools.partial(
    pl.kernel,
    out_type=jax.ShapeDtypeStruct((NPAD, EMB), jnp.float32),
    mesh=_mesh,
    scratch_types=[
        pltpu.VMEM((CH,), jnp.int32),
        pltpu.VMEM((CH,), jnp.int32),
        pltpu.VMEM((CH, EMB), jnp.float32),
        pltpu.VMEM((CH, EMB), jnp.float32),
        pltpu.SemaphoreType.DMA,
        pltpu.SemaphoreType.DMA,
    ],
)
def _embed_sc(keys_hbm, vals_hbm, xk_hbm, xv_hbm, out_hbm,
              idk, idv, bufk, bufv, semk, semv):
    c = lax.axis_index("c")
    s = lax.axis_index("s")
    wid = c * NS + s
    base0 = wid * EMB_PW

    def chunk(j, carry):
        base = base0 + j * CH
        pltpu.sync_copy(xk_hbm.at[pl.ds(base, CH)], idk)
        pltpu.sync_copy(xv_hbm.at[pl.ds(base, CH)], idv)
        ck = pltpu.async_copy(keys_hbm.at[idk], bufk, semk)
        cv = pltpu.async_copy(vals_hbm.at[idv], bufv, semv)
        ck.wait()
        cv.wait()

        def rbody(r, carry2):
            for q in range(EMB // 16):
                sl = (r, pl.ds(q * 16, 16))
                bufk[sl] = bufk[sl] + bufv[sl]
            return carry2

        lax.fori_loop(0, CH, rbody, 0)
        pltpu.sync_copy(bufk, out_hbm.at[pl.ds(base, CH)])
        return carry

    lax.fori_loop(0, EMB_PW // CH, chunk, 0)


@functools.partial(
    pl.kernel,
    out_type=jax.ShapeDtypeStruct((NC, NAPAD, EMB), jnp.float32),
    mesh=_mesh,
    scratch_types=[
        pltpu.VMEM((CH,), jnp.int32),
        pltpu.VMEM((CH,), jnp.int32),
        pltpu.VMEM((CH,), jnp.int32),
        pltpu.VMEM((CH,), jnp.int32),
        pltpu.VMEM((CH, EMB), jnp.float32),
        pltpu.VMEM_SHARED((NAPAD, EMB), jnp.float32),
        pltpu.SemaphoreType.DMA,
    ],
)
def _message_sc(hplus_hbm, src_hbm, attr_hbm, dst_hbm, zeros_hbm, out_hbm,
                gsrc, gattr, gidx, gdst, rows, aggr_sh, sem):
    c = lax.axis_index("c")
    s = lax.axis_index("s")
    wid = c * NS + s
    r0 = s * ROWS_PT
    pltpu.sync_copy(zeros_hbm.at[pl.ds(r0, ROWS_PT)], aggr_sh.at[pl.ds(r0, ROWS_PT)])
    plsc.subcore_barrier()

    base0 = wid * EPW

    def chunk(j, carry):
        base = base0 + j * CH
        pltpu.sync_copy(src_hbm.at[pl.ds(base, CH)], gsrc)
        pltpu.sync_copy(attr_hbm.at[pl.ds(base, CH)], gattr)
        pltpu.sync_copy(dst_hbm.at[pl.ds(base, CH)], gdst)
        for k in range(CH // 16):
            sl = pl.ds(k * 16, 16)
            gidx[sl] = gattr[sl] * N + gsrc[sl]
        pltpu.async_copy(hplus_hbm.at[gidx], rows, sem).wait()

        def rbody(r, carry2):
            for q in range(EMB // 16):
                sl = (r, pl.ds(q * 16, 16))
                rows[sl] = jnp.maximum(rows[sl], 0.0)
            return carry2

        lax.fori_loop(0, CH, rbody, 0)
        pltpu.sync_copy(rows, aggr_sh.at[gdst], add=True)
        return carry

    lax.fori_loop(0, NCHUNK, chunk, 0)
    plsc.subcore_barrier()
    pltpu.sync_copy(aggr_sh.at[pl.ds(r0, ROWS_PT)], out_hbm.at[c, pl.ds(r0, ROWS_PT)])


def _hplus_body(h_ref, et_ref, out_ref):
    out_ref[...] = h_ref[...][None, :, :] + et_ref[...][:, None, :]


_BLK = 1000
_NB = N // _BLK


def _build_hplus(h, et):
    return pl.pallas_call(
        _hplus_body,
        grid=(_NB,),
        in_specs=[
            pl.BlockSpec((_BLK, EMB), lambda i: (i, 0)),
            pl.BlockSpec((NTYPES, EMB), lambda i: (0, 0)),
        ],
        out_specs=pl.BlockSpec((NTYPES, _BLK, EMB), lambda i: (0, i, 0)),
        out_shape=jax.ShapeDtypeStruct((NTYPES, N, EMB), jnp.float32),
    )(h, et)


def _mlp_body(eps_ref, h_ref, ag_ref, w1_ref, b1_ref, w2_ref, b2_ref,
              y_ref, cs_ref, cq_ref):
    i = pl.program_id(0)
    z = (1.0 + eps_ref[0]) * h_ref[...] + ag_ref[0] + ag_ref[1]
    t = jnp.dot(z, w1_ref[...], preferred_element_type=jnp.float32) + b1_ref[...]
    t = jnp.maximum(t, 0.0)
    y = jnp.dot(t, w2_ref[...], preferred_element_type=jnp.float32) + b2_ref[...]
    y_ref[...] = y

    @pl.when(i == 0)
    def _():
        cs_ref[...] = jnp.zeros_like(cs_ref)
        cq_ref[...] = jnp.zeros_like(cq_ref)

    cs_ref[...] += jnp.sum(y, axis=0, keepdims=True)
    cq_ref[...] += jnp.sum(y * y, axis=0, keepdims=True)


def _mlp_stats(h, aggr2, w1, b1, w2, b2, eps):
    return pl.pallas_call(
        _mlp_body,
        grid=(_NB,),
        in_specs=[
            pl.BlockSpec(memory_space=pltpu.SMEM),
            pl.BlockSpec((_BLK, EMB), lambda i: (i, 0)),
            pl.BlockSpec((NC, _BLK, EMB), lambda i: (0, i, 0)),
            pl.BlockSpec((EMB, 2 * EMB), lambda i: (0, 0)),
            pl.BlockSpec((1, 2 * EMB), lambda i: (0, 0)),
            pl.BlockSpec((2 * EMB, EMB), lambda i: (0, 0)),
            pl.BlockSpec((1, EMB), lambda i: (0, 0)),
        ],
        out_specs=[
            pl.BlockSpec((_BLK, EMB), lambda i: (i, 0)),
            pl.BlockSpec((1, EMB), lambda i: (0, 0)),
            pl.BlockSpec((1, EMB), lambda i: (0, 0)),
        ],
        out_shape=[
            jax.ShapeDtypeStruct((N, EMB), jnp.float32),
            jax.ShapeDtypeStruct((1, EMB), jnp.float32),
            jax.ShapeDtypeStruct((1, EMB), jnp.float32),
        ],
    )(eps, h, aggr2, w1, b1, w2, b2)


def _norm_body(y_ref, h_ref, sc_ref, sh_ref, lg_ref, lb_ref, out_ref):
    z = y_ref[...] * sc_ref[...] + sh_ref[...]
    z = jnp.maximum(z, 0.0) + h_ref[...]
    m = jnp.mean(z, axis=1, keepdims=True)
    zc = z - m
    v = jnp.mean(zc * zc, axis=1, keepdims=True)
    out_ref[...] = zc * lax.rsqrt(v + 1e-5) * lg_ref[...] + lb_ref[...]


def _norm_hplus_body(y_ref, h_ref, sc_ref, sh_ref, lg_ref, lb_ref, et_ref,
                     out_ref, hp_ref):
    z = y_ref[...] * sc_ref[...] + sh_ref[...]
    z = jnp.maximum(z, 0.0) + h_ref[...]
    m = jnp.mean(z, axis=1, keepdims=True)
    zc = z - m
    v = jnp.mean(zc * zc, axis=1, keepdims=True)
    hn = zc * lax.rsqrt(v + 1e-5) * lg_ref[...] + lb_ref[...]
    out_ref[...] = hn
    hp_ref[...] = hn[None, :, :] + et_ref[...][:, None, :]


_ROW_SPECS = [
    pl.BlockSpec((_BLK, EMB), lambda i: (i, 0)),
    pl.BlockSpec((_BLK, EMB), lambda i: (i, 0)),
    pl.BlockSpec((1, EMB), lambda i: (0, 0)),
    pl.BlockSpec((1, EMB), lambda i: (0, 0)),
    pl.BlockSpec((1, EMB), lambda i: (0, 0)),
    pl.BlockSpec((1, EMB), lambda i: (0, 0)),
]


def _norm(y, h, scale, shift, lg, lb):
    return pl.pallas_call(
        _norm_body,
        grid=(_NB,),
        in_specs=_ROW_SPECS,
        out_specs=pl.BlockSpec((_BLK, EMB), lambda i: (i, 0)),
        out_shape=jax.ShapeDtypeStruct((N, EMB), jnp.float32),
    )(y, h, scale, shift, lg, lb)


def _norm_hplus(y, h, scale, shift, lg, lb, et):
    return pl.pallas_call(
        _norm_hplus_body,
        grid=(_NB,),
        in_specs=_ROW_SPECS + [pl.BlockSpec((NTYPES, EMB), lambda i: (0, 0))],
        out_specs=[
            pl.BlockSpec((_BLK, EMB), lambda i: (i, 0)),
            pl.BlockSpec((NTYPES, _BLK, EMB), lambda i: (0, i, 0)),
        ],
        out_shape=[
            jax.ShapeDtypeStruct((N, EMB), jnp.float32),
            jax.ShapeDtypeStruct((NTYPES, N, EMB), jnp.float32),
        ],
    )(y, h, scale, shift, lg, lb, et)


def kernel(x, edge_index, edge_attr, batch, root_mask, params):
    x = x.astype(jnp.int32)
    src = edge_index[0].astype(jnp.int32)
    dst = edge_index[1].astype(jnp.int32)
    attr = edge_attr.astype(jnp.int32)
    pad = jnp.zeros((NPAD - N,), jnp.int32)
    xk = jnp.concatenate([x[:, 0], pad])
    xv = jnp.concatenate([x[:, 1], pad])

    h = _embed_sc(params['keys_table'], params['values_table'], xk, xv)[:N]
    zeros = jnp.zeros((NAPAD, EMB), jnp.float32)

    layers = params['layers']
    hplus = None
    for l, lp in enumerate(layers):
        if hplus is None:
            hplus = _build_hplus(h, lp['edge_table'])
        aggr2 = _message_sc(hplus.reshape(NTYPES * N, EMB), src, attr, dst, zeros)
        y, cs, cq = _mlp_stats(h, aggr2,
                               lp['W1'], lp['b1'].reshape(1, -1),
                               lp['W2'], lp['b2'].reshape(1, -1),
                               lp['eps'].reshape(1))
        mean = cs / N
        var = cq / N - mean * mean
        scale = lp['bn_g'].reshape(1, EMB) * lax.rsqrt(var + 1e-5)
        shift = lp['bn_b'].reshape(1, EMB) - mean * scale
        lg = lp['ln_g'].reshape(1, EMB)
        lb = lp['ln_b'].reshape(1, EMB)
        if l + 1 < len(layers):
            h, hplus = _norm_hplus(y, h, scale, shift, lg, lb,
                                   layers[l + 1]['edge_table'])
        else:
            h = _norm(y, h, scale, shift, lg, lb)
    return h


# speedup vs baseline: 9.2929x; 2.1765x over previous
import functools

import jax
import jax.numpy as jnp
from jax import lax
from jax.experimental import pallas as pl
from jax.experimental.pallas import tpu as pltpu
from jax.experimental.pallas import tpu_sc as plsc

N = 10000
EMB = 128
E = 320000
NTYPES = 4

NC, NS = 2, 16
NW = NC * NS
CH = 80
EPW = E // NW
NCHUNK = EPW // CH
NPAD = 10240
EMB_PW = NPAD // NW
NAPAD = 10112
ROWS_PT = NAPAD // NS

_mesh = plsc.VectorSubcoreMesh(core_axis_name="c", subcore_axis_name="s")


@functools.partial(
    pl.kernel,
    out_type=jax.ShapeDtypeStruct((NPAD, EMB), jnp.float32),
    mesh=_mesh,
    scratch_types=[
        pltpu.VMEM((CH,), jnp.int32),
        pltpu.VMEM((CH,), jnp.int32),
        pltpu.VMEM((CH, EMB), jnp.float32),
        pltpu.VMEM((CH, EMB), jnp.float32),
        pltpu.SemaphoreType.DMA,
        pltpu.SemaphoreType.DMA,
    ],
)
def _embed_sc(keys_hbm, vals_hbm, xk_hbm, xv_hbm, out_hbm,
              idk, idv, bufk, bufv, semk, semv):
    c = lax.axis_index("c")
    s = lax.axis_index("s")
    wid = c * NS + s
    base0 = wid * EMB_PW

    def chunk(j, carry):
        base = base0 + j * CH
        pltpu.sync_copy(xk_hbm.at[pl.ds(base, CH)], idk)
        pltpu.sync_copy(xv_hbm.at[pl.ds(base, CH)], idv)
        ck = pltpu.async_copy(keys_hbm.at[idk], bufk, semk)
        cv = pltpu.async_copy(vals_hbm.at[idv], bufv, semv)
        ck.wait()
        cv.wait()

        def rbody(r, carry2):
            for q in range(EMB // 16):
                sl = (r, pl.ds(q * 16, 16))
                bufk[sl] = bufk[sl] + bufv[sl]
            return carry2

        lax.fori_loop(0, CH, rbody, 0)
        pltpu.sync_copy(bufk, out_hbm.at[pl.ds(base, CH)])
        return carry

    lax.fori_loop(0, EMB_PW // CH, chunk, 0)


@functools.partial(
    pl.kernel,
    out_type=jax.ShapeDtypeStruct((NC, NAPAD, EMB), jnp.float32),
    mesh=_mesh,
    scratch_types=[
        pltpu.VMEM((EPW,), jnp.int32),
        pltpu.VMEM((EPW,), jnp.int32),
        pltpu.VMEM((CH,), jnp.int32),
        pltpu.VMEM((CH,), jnp.int32),
        pltpu.VMEM((CH, EMB), jnp.float32),
        pltpu.VMEM((CH, EMB), jnp.float32),
        pltpu.VMEM_SHARED((NAPAD, EMB), jnp.float32),
        pltpu.SemaphoreType.DMA,
        pltpu.SemaphoreType.DMA,
        pltpu.SemaphoreType.DMA,
        pltpu.SemaphoreType.DMA,
    ],
)
def _message_sc(hplus_hbm, gidx_hbm, dst_hbm, zeros_hbm, out_hbm,
                gidxf, dstf, dstb0, dstb1, rows0, rows1, aggr_sh,
                gsem0, gsem1, ssem0, ssem1):
    c = lax.axis_index("c")
    s = lax.axis_index("s")
    wid = c * NS + s
    r0 = s * ROWS_PT
    pltpu.sync_copy(zeros_hbm.at[pl.ds(r0, ROWS_PT)], aggr_sh.at[pl.ds(r0, ROWS_PT)])

    pltpu.sync_copy(gidx_hbm.at[wid], gidxf)
    pltpu.sync_copy(dst_hbm.at[wid], dstf)
    plsc.subcore_barrier()

    def fire_gather(j, rows_b, sem_b):
        pltpu.async_copy(hplus_hbm.at[gidxf.at[pl.ds(j * CH, CH)]], rows_b, sem_b)

    def wait_gather(j, rows_b, sem_b):
        pltpu.make_async_copy(hplus_hbm.at[gidxf.at[pl.ds(j * CH, CH)]],
                              rows_b, sem_b).wait()

    def fill_dstb(j, dstb):
        for k in range(CH // 16):
            dstb[pl.ds(k * 16, 16)] = dstf[pl.ds(j * CH + k * 16, 16)]

    def fire_scatter(rows_b, dstb, sem_b):
        pltpu.async_copy(rows_b, aggr_sh.at[dstb], sem_b, add=True)

    def wait_scatter(rows_b, dstb, sem_b):
        pltpu.make_async_copy(rows_b, aggr_sh.at[dstb], sem_b).wait()

    fire_gather(0, rows0, gsem0)
    fire_gather(1, rows1, gsem1)

    def body(i, carry):
        j0 = 2 * i
        j1 = j0 + 1
        wait_gather(j0, rows0, gsem0)
        fill_dstb(j0, dstb0)
        fire_scatter(rows0, dstb0, ssem0)
        wait_gather(j1, rows1, gsem1)
        fill_dstb(j1, dstb1)
        fire_scatter(rows1, dstb1, ssem1)
        wait_scatter(rows0, dstb0, ssem0)

        @pl.when(j0 + 2 < NCHUNK)
        def _():
            fire_gather(j0 + 2, rows0, gsem0)

        wait_scatter(rows1, dstb1, ssem1)

        @pl.when(j1 + 2 < NCHUNK)
        def _():
            fire_gather(j1 + 2, rows1, gsem1)

        return carry

    lax.fori_loop(0, NCHUNK // 2, body, 0)
    if NCHUNK % 2:
        jl = NCHUNK - 1
        wait_gather(jl, rows0, gsem0)
        fill_dstb(jl, dstb0)
        fire_scatter(rows0, dstb0, ssem0)
        wait_scatter(rows0, dstb0, ssem0)
    plsc.subcore_barrier()
    pltpu.sync_copy(aggr_sh.at[pl.ds(r0, ROWS_PT)], out_hbm.at[c, pl.ds(r0, ROWS_PT)])


def _hplus_body(h_ref, et_ref, out_ref):
    out_ref[...] = jnp.maximum(h_ref[...][None, :, :] + et_ref[...][:, None, :],
                               0.0)


_BLK = 1000
_NB = N // _BLK


def _build_hplus(h, et):
    return pl.pallas_call(
        _hplus_body,
        grid=(_NB,),
        in_specs=[
            pl.BlockSpec((_BLK, EMB), lambda i: (i, 0)),
            pl.BlockSpec((NTYPES, EMB), lambda i: (0, 0)),
        ],
        out_specs=pl.BlockSpec((NTYPES, _BLK, EMB), lambda i: (0, i, 0)),
        out_shape=jax.ShapeDtypeStruct((NTYPES, N, EMB), jnp.float32),
    )(h, et)


def _mlp_body(eps_ref, h_ref, ag_ref, w1_ref, b1_ref, w2_ref, b2_ref,
              y_ref, cs_ref, cq_ref):
    i = pl.program_id(0)
    z = (1.0 + eps_ref[0]) * h_ref[...] + ag_ref[0] + ag_ref[1]
    t = jnp.dot(z, w1_ref[...], preferred_element_type=jnp.float32) + b1_ref[...]
    t = jnp.maximum(t, 0.0)
    y = jnp.dot(t, w2_ref[...], preferred_element_type=jnp.float32) + b2_ref[...]
    y_ref[...] = y

    @pl.when(i == 0)
    def _():
        cs_ref[...] = jnp.zeros_like(cs_ref)
        cq_ref[...] = jnp.zeros_like(cq_ref)

    cs_ref[...] += jnp.sum(y, axis=0, keepdims=True)
    cq_ref[...] += jnp.sum(y * y, axis=0, keepdims=True)


def _mlp_stats(h, aggr2, w1, b1, w2, b2, eps):
    return pl.pallas_call(
        _mlp_body,
        grid=(_NB,),
        in_specs=[
            pl.BlockSpec(memory_space=pltpu.SMEM),
            pl.BlockSpec((_BLK, EMB), lambda i: (i, 0)),
            pl.BlockSpec((NC, _BLK, EMB), lambda i: (0, i, 0)),
            pl.BlockSpec((EMB, 2 * EMB), lambda i: (0, 0)),
            pl.BlockSpec((1, 2 * EMB), lambda i: (0, 0)),
            pl.BlockSpec((2 * EMB, EMB), lambda i: (0, 0)),
            pl.BlockSpec((1, EMB), lambda i: (0, 0)),
        ],
        out_specs=[
            pl.BlockSpec((_BLK, EMB), lambda i: (i, 0)),
            pl.BlockSpec((1, EMB), lambda i: (0, 0)),
            pl.BlockSpec((1, EMB), lambda i: (0, 0)),
        ],
        out_shape=[
            jax.ShapeDtypeStruct((N, EMB), jnp.float32),
            jax.ShapeDtypeStruct((1, EMB), jnp.float32),
            jax.ShapeDtypeStruct((1, EMB), jnp.float32),
        ],
    )(eps, h, aggr2, w1, b1, w2, b2)


def _norm_body(y_ref, h_ref, sc_ref, sh_ref, lg_ref, lb_ref, out_ref):
    z = y_ref[...] * sc_ref[...] + sh_ref[...]
    z = jnp.maximum(z, 0.0) + h_ref[...]
    m = jnp.mean(z, axis=1, keepdims=True)
    zc = z - m
    v = jnp.mean(zc * zc, axis=1, keepdims=True)
    out_ref[...] = zc * lax.rsqrt(v + 1e-5) * lg_ref[...] + lb_ref[...]


def _norm_hplus_body(y_ref, h_ref, sc_ref, sh_ref, lg_ref, lb_ref, et_ref,
                     out_ref, hp_ref):
    z = y_ref[...] * sc_ref[...] + sh_ref[...]
    z = jnp.maximum(z, 0.0) + h_ref[...]
    m = jnp.mean(z, axis=1, keepdims=True)
    zc = z - m
    v = jnp.mean(zc * zc, axis=1, keepdims=True)
    hn = zc * lax.rsqrt(v + 1e-5) * lg_ref[...] + lb_ref[...]
    out_ref[...] = hn
    hp_ref[...] = jnp.maximum(hn[None, :, :] + et_ref[...][:, None, :], 0.0)


_ROW_SPECS = [
    pl.BlockSpec((_BLK, EMB), lambda i: (i, 0)),
    pl.BlockSpec((_BLK, EMB), lambda i: (i, 0)),
    pl.BlockSpec((1, EMB), lambda i: (0, 0)),
    pl.BlockSpec((1, EMB), lambda i: (0, 0)),
    pl.BlockSpec((1, EMB), lambda i: (0, 0)),
    pl.BlockSpec((1, EMB), lambda i: (0, 0)),
]


def _norm(y, h, scale, shift, lg, lb):
    return pl.pallas_call(
        _norm_body,
        grid=(_NB,),
        in_specs=_ROW_SPECS,
        out_specs=pl.BlockSpec((_BLK, EMB), lambda i: (i, 0)),
        out_shape=jax.ShapeDtypeStruct((N, EMB), jnp.float32),
    )(y, h, scale, shift, lg, lb)


def _norm_hplus(y, h, scale, shift, lg, lb, et):
    return pl.pallas_call(
        _norm_hplus_body,
        grid=(_NB,),
        in_specs=_ROW_SPECS + [pl.BlockSpec((NTYPES, EMB), lambda i: (0, 0))],
        out_specs=[
            pl.BlockSpec((_BLK, EMB), lambda i: (i, 0)),
            pl.BlockSpec((NTYPES, _BLK, EMB), lambda i: (0, i, 0)),
        ],
        out_shape=[
            jax.ShapeDtypeStruct((N, EMB), jnp.float32),
            jax.ShapeDtypeStruct((NTYPES, N, EMB), jnp.float32),
        ],
    )(y, h, scale, shift, lg, lb, et)


def kernel(x, edge_index, edge_attr, batch, root_mask, params):
    x = x.astype(jnp.int32)
    src = edge_index[0].astype(jnp.int32)
    dst = edge_index[1].astype(jnp.int32)
    attr = edge_attr.astype(jnp.int32)
    pad = jnp.zeros((NPAD - N,), jnp.int32)
    xk = jnp.concatenate([x[:, 0], pad])
    xv = jnp.concatenate([x[:, 1], pad])

    h = _embed_sc(params['keys_table'], params['values_table'], xk, xv)[:N]
    zeros = jnp.zeros((NAPAD, EMB), jnp.float32)
    gidx = (attr * N + src).reshape(NW, EPW)
    dst2 = dst.reshape(NW, EPW)

    layers = params['layers']
    hplus = None
    for l, lp in enumerate(layers):
        if hplus is None:
            hplus = _build_hplus(h, lp['edge_table'])
        aggr2 = _message_sc(hplus.reshape(NTYPES * N, EMB), gidx, dst2, zeros)
        y, cs, cq = _mlp_stats(h, aggr2,
                               lp['W1'], lp['b1'].reshape(1, -1),
                               lp['W2'], lp['b2'].reshape(1, -1),
                               lp['eps'].reshape(1))
        mean = cs / N
        var = cq / N - mean * mean
        scale = lp['bn_g'].reshape(1, EMB) * lax.rsqrt(var + 1e-5)
        shift = lp['bn_b'].reshape(1, EMB) - mean * scale
        lg = lp['ln_g'].reshape(1, EMB)
        lb = lp['ln_b'].reshape(1, EMB)
        if l + 1 < len(layers):
            h, hplus = _norm_hplus(y, h, scale, shift, lg, lb,
                                   layers[l + 1]['edge_table'])
        else:
            h = _norm(y, h, scale, shift, lg, lb)
    return h


# speedup vs baseline: 12.6892x; 1.3655x over previous
import functools

import jax
import jax.numpy as jnp
from jax import lax
from jax.experimental import pallas as pl
from jax.experimental.pallas import tpu as pltpu
from jax.experimental.pallas import tpu_sc as plsc

N = 10000
EMB = 128
E = 320000
NTYPES = 4

NC, NS = 2, 16
NW = NC * NS
CH = 80
EPW = E // NW
NCHUNK = EPW // CH
NPAD = 10240
EMB_PW = NPAD // NW
NAPAD = 10112
ROWS_PT = NAPAD // NS

_mesh = plsc.VectorSubcoreMesh(core_axis_name="c", subcore_axis_name="s")


@functools.partial(
    pl.kernel,
    out_type=jax.ShapeDtypeStruct((NPAD, EMB), jnp.float32),
    mesh=_mesh,
    scratch_types=[
        pltpu.VMEM((CH,), jnp.int32),
        pltpu.VMEM((CH,), jnp.int32),
        pltpu.VMEM((CH, EMB), jnp.float32),
        pltpu.VMEM((CH, EMB), jnp.float32),
        pltpu.SemaphoreType.DMA,
        pltpu.SemaphoreType.DMA,
    ],
)
def _embed_sc(keys_hbm, vals_hbm, xk_hbm, xv_hbm, out_hbm,
              idk, idv, bufk, bufv, semk, semv):
    c = lax.axis_index("c")
    s = lax.axis_index("s")
    wid = c * NS + s
    base0 = wid * EMB_PW

    def chunk(j, carry):
        base = base0 + j * CH
        pltpu.sync_copy(xk_hbm.at[pl.ds(base, CH)], idk)
        pltpu.sync_copy(xv_hbm.at[pl.ds(base, CH)], idv)
        ck = pltpu.async_copy(keys_hbm.at[idk], bufk, semk)
        cv = pltpu.async_copy(vals_hbm.at[idv], bufv, semv)
        ck.wait()
        cv.wait()

        def rbody(r, carry2):
            for q in range(EMB // 16):
                sl = (r, pl.ds(q * 16, 16))
                bufk[sl] = bufk[sl] + bufv[sl]
            return carry2

        lax.fori_loop(0, CH, rbody, 0)
        pltpu.sync_copy(bufk, out_hbm.at[pl.ds(base, CH)])
        return carry

    lax.fori_loop(0, EMB_PW // CH, chunk, 0)


@functools.partial(
    pl.kernel,
    out_type=jax.ShapeDtypeStruct((NC, NAPAD, EMB), jnp.float32),
    mesh=_mesh,
    scratch_types=(
        [pltpu.VMEM((CH,), jnp.int32)] * 4
        + [pltpu.VMEM((CH,), jnp.int32)] * 4
        + [pltpu.VMEM((CH, EMB), jnp.float32)] * 4
        + [pltpu.SemaphoreType.DMA] * 12
        + [pltpu.VMEM_SHARED((NAPAD, EMB), jnp.float32)]
    ),
)
def _message_sc(hplus_hbm, gidx_hbm, dst_hbm, zeros_hbm, out_hbm,
                gb0, gb1, gb2, gb3, db0, db1, db2, db3,
                rw0, rw1, rw2, rw3,
                is0, is1, is2, is3, gs0, gs1, gs2, gs3, ss0, ss1, ss2, ss3,
                aggr_sh):
    GB = (gb0, gb1, gb2, gb3)
    DB = (db0, db1, db2, db3)
    RW = (rw0, rw1, rw2, rw3)
    IS = (is0, is1, is2, is3)
    GS = (gs0, gs1, gs2, gs3)
    SS = (ss0, ss1, ss2, ss3)
    c = lax.axis_index("c")
    s = lax.axis_index("s")
    wid = c * NS + s
    r0 = s * ROWS_PT

    base0 = wid * EPW

    def fire_idx(j, b):
        pltpu.async_copy(gidx_hbm.at[pl.ds(base0 + j * CH, CH)], GB[b], IS[b])
        pltpu.async_copy(dst_hbm.at[pl.ds(base0 + j * CH, CH)], DB[b], IS[b])

    def wait_idx(j, b):
        pltpu.make_async_copy(gidx_hbm.at[pl.ds(base0 + j * CH, CH)], GB[b], IS[b]).wait()
        pltpu.make_async_copy(dst_hbm.at[pl.ds(base0 + j * CH, CH)], DB[b], IS[b]).wait()

    def fire_gather(b):
        pltpu.async_copy(hplus_hbm.at[GB[b]], RW[b], GS[b])

    def wait_gather(b):
        pltpu.make_async_copy(hplus_hbm.at[GB[b]], RW[b], GS[b]).wait()

    def fire_scatter(b):
        pltpu.async_copy(RW[b], aggr_sh.at[DB[b]], SS[b], add=True)

    def wait_scatter(b):
        pltpu.make_async_copy(RW[b], aggr_sh.at[DB[b]], SS[b]).wait()

    fire_idx(0, 0)
    fire_idx(1, 1)
    fire_idx(2, 2)
    pltpu.sync_copy(zeros_hbm.at[pl.ds(r0, ROWS_PT)], aggr_sh.at[pl.ds(r0, ROWS_PT)])
    wait_idx(0, 0)
    fire_gather(0)
    wait_idx(1, 1)
    fire_gather(1)
    plsc.subcore_barrier()

    def process(j, k):
        wait_gather(k)
        fire_scatter(k)
        b3 = (k + 3) % 4

        @pl.when(j >= 1)
        def _():
            wait_scatter(b3)

        @pl.when(j + 3 < NCHUNK)
        def _():
            fire_idx(j + 3, b3)

        b2 = (k + 2) % 4

        @pl.when(j + 2 < NCHUNK)
        def _():
            wait_idx(j + 2, b2)
            fire_gather(b2)

    def body(i, carry):
        j = 4 * i
        for k in range(4):
            process(j + k, k)
        return carry

    lax.fori_loop(0, NCHUNK // 4, body, 0)
    for jl in range(NCHUNK - NCHUNK % 4, NCHUNK):
        process(jl, jl % 4)
    wait_scatter((NCHUNK - 1) % 4)
    plsc.subcore_barrier()
    pltpu.sync_copy(aggr_sh.at[pl.ds(r0, ROWS_PT)], out_hbm.at[c, pl.ds(r0, ROWS_PT)])


def _hplus_body(h_ref, et_ref, out_ref):
    out_ref[...] = jnp.maximum(h_ref[...][None, :, :] + et_ref[...][:, None, :],
                               0.0)


_BLK = 1000
_NB = N // _BLK


def _build_hplus(h, et):
    return pl.pallas_call(
        _hplus_body,
        grid=(_NB,),
        in_specs=[
            pl.BlockSpec((_BLK, EMB), lambda i: (i, 0)),
            pl.BlockSpec((NTYPES, EMB), lambda i: (0, 0)),
        ],
        out_specs=pl.BlockSpec((NTYPES, _BLK, EMB), lambda i: (0, i, 0)),
        out_shape=jax.ShapeDtypeStruct((NTYPES, N, EMB), jnp.float32),
    )(h, et)


def _mlp_body(eps_ref, h_ref, ag_ref, w1_ref, b1_ref, w2_ref, b2_ref,
              y_ref, cs_ref, cq_ref):
    i = pl.program_id(0)
    z = (1.0 + eps_ref[0]) * h_ref[...] + ag_ref[0] + ag_ref[1]
    t = jnp.dot(z, w1_ref[...], preferred_element_type=jnp.float32) + b1_ref[...]
    t = jnp.maximum(t, 0.0)
    y = jnp.dot(t, w2_ref[...], preferred_element_type=jnp.float32) + b2_ref[...]
    y_ref[...] = y

    @pl.when(i == 0)
    def _():
        cs_ref[...] = jnp.zeros_like(cs_ref)
        cq_ref[...] = jnp.zeros_like(cq_ref)

    cs_ref[...] += jnp.sum(y, axis=0, keepdims=True)
    cq_ref[...] += jnp.sum(y * y, axis=0, keepdims=True)


def _mlp_stats(h, aggr2, w1, b1, w2, b2, eps):
    return pl.pallas_call(
        _mlp_body,
        grid=(_NB,),
        in_specs=[
            pl.BlockSpec(memory_space=pltpu.SMEM),
            pl.BlockSpec((_BLK, EMB), lambda i: (i, 0)),
            pl.BlockSpec((NC, _BLK, EMB), lambda i: (0, i, 0)),
            pl.BlockSpec((EMB, 2 * EMB), lambda i: (0, 0)),
            pl.BlockSpec((1, 2 * EMB), lambda i: (0, 0)),
            pl.BlockSpec((2 * EMB, EMB), lambda i: (0, 0)),
            pl.BlockSpec((1, EMB), lambda i: (0, 0)),
        ],
        out_specs=[
            pl.BlockSpec((_BLK, EMB), lambda i: (i, 0)),
            pl.BlockSpec((1, EMB), lambda i: (0, 0)),
            pl.BlockSpec((1, EMB), lambda i: (0, 0)),
        ],
        out_shape=[
            jax.ShapeDtypeStruct((N, EMB), jnp.float32),
            jax.ShapeDtypeStruct((1, EMB), jnp.float32),
            jax.ShapeDtypeStruct((1, EMB), jnp.float32),
        ],
    )(eps, h, aggr2, w1, b1, w2, b2)


def _norm_body(y_ref, h_ref, sc_ref, sh_ref, lg_ref, lb_ref, out_ref):
    z = y_ref[...] * sc_ref[...] + sh_ref[...]
    z = jnp.maximum(z, 0.0) + h_ref[...]
    m = jnp.mean(z, axis=1, keepdims=True)
    zc = z - m
    v = jnp.mean(zc * zc, axis=1, keepdims=True)
    out_ref[...] = zc * lax.rsqrt(v + 1e-5) * lg_ref[...] + lb_ref[...]


def _norm_hplus_body(y_ref, h_ref, sc_ref, sh_ref, lg_ref, lb_ref, et_ref,
                     out_ref, hp_ref):
    z = y_ref[...] * sc_ref[...] + sh_ref[...]
    z = jnp.maximum(z, 0.0) + h_ref[...]
    m = jnp.mean(z, axis=1, keepdims=True)
    zc = z - m
    v = jnp.mean(zc * zc, axis=1, keepdims=True)
    hn = zc * lax.rsqrt(v + 1e-5) * lg_ref[...] + lb_ref[...]
    out_ref[...] = hn
    hp_ref[...] = jnp.maximum(hn[None, :, :] + et_ref[...][:, None, :], 0.0)


_ROW_SPECS = [
    pl.BlockSpec((_BLK, EMB), lambda i: (i, 0)),
    pl.BlockSpec((_BLK, EMB), lambda i: (i, 0)),
    pl.BlockSpec((1, EMB), lambda i: (0, 0)),
    pl.BlockSpec((1, EMB), lambda i: (0, 0)),
    pl.BlockSpec((1, EMB), lambda i: (0, 0)),
    pl.BlockSpec((1, EMB), lambda i: (0, 0)),
]


def _norm(y, h, scale, shift, lg, lb):
    return pl.pallas_call(
        _norm_body,
        grid=(_NB,),
        in_specs=_ROW_SPECS,
        out_specs=pl.BlockSpec((_BLK, EMB), lambda i: (i, 0)),
        out_shape=jax.ShapeDtypeStruct((N, EMB), jnp.float32),
    )(y, h, scale, shift, lg, lb)


def _norm_hplus(y, h, scale, shift, lg, lb, et):
    return pl.pallas_call(
        _norm_hplus_body,
        grid=(_NB,),
        in_specs=_ROW_SPECS + [pl.BlockSpec((NTYPES, EMB), lambda i: (0, 0))],
        out_specs=[
            pl.BlockSpec((_BLK, EMB), lambda i: (i, 0)),
            pl.BlockSpec((NTYPES, _BLK, EMB), lambda i: (0, i, 0)),
        ],
        out_shape=[
            jax.ShapeDtypeStruct((N, EMB), jnp.float32),
            jax.ShapeDtypeStruct((NTYPES, N, EMB), jnp.float32),
        ],
    )(y, h, scale, shift, lg, lb, et)


def kernel(x, edge_index, edge_attr, batch, root_mask, params):
    x = x.astype(jnp.int32)
    src = edge_index[0].astype(jnp.int32)
    dst = edge_index[1].astype(jnp.int32)
    attr = edge_attr.astype(jnp.int32)
    pad = jnp.zeros((NPAD - N,), jnp.int32)
    xk = jnp.concatenate([x[:, 0], pad])
    xv = jnp.concatenate([x[:, 1], pad])

    h = _embed_sc(params['keys_table'], params['values_table'], xk, xv)[:N]
    zeros = jnp.zeros((NAPAD, EMB), jnp.float32)
    gidx = attr * N + src
    dst2 = dst

    layers = params['layers']
    hplus = None
    for l, lp in enumerate(layers):
        if hplus is None:
            hplus = _build_hplus(h, lp['edge_table'])
        aggr2 = _message_sc(hplus.reshape(NTYPES * N, EMB), gidx, dst2, zeros)
        y, cs, cq = _mlp_stats(h, aggr2,
                               lp['W1'], lp['b1'].reshape(1, -1),
                               lp['W2'], lp['b2'].reshape(1, -1),
                               lp['eps'].reshape(1))
        mean = cs / N
        var = cq / N - mean * mean
        scale = lp['bn_g'].reshape(1, EMB) * lax.rsqrt(var + 1e-5)
        shift = lp['bn_b'].reshape(1, EMB) - mean * scale
        lg = lp['ln_g'].reshape(1, EMB)
        lb = lp['ln_b'].reshape(1, EMB)
        if l + 1 < len(layers):
            h, hplus = _norm_hplus(y, h, scale, shift, lg, lb,
                                   layers[l + 1]['edge_table'])
        else:
            h = _norm(y, h, scale, shift, lg, lb)
    return h
